# baseline (speedup 1.0000x reference)
import functools

import jax
import jax.numpy as jnp
from jax import lax
from jax.experimental import pallas as pl
from jax.experimental.pallas import tpu as pltpu
from jax.experimental.pallas import tpu_sc as plsc

D_MODEL = 32
NC = 2
NS = 16
NW = NC * NS
JC = 4
IT = 128
L = 16


@functools.lru_cache(maxsize=None)
def _make_depad(V: int):
    rows = 256
    n_chunks = V // rows
    tail_base = n_chunks * rows
    tail_rows = ((V - tail_base) // 8) * 8
    last_row = V - 1
    vp4 = ((V // 4) // 8 + 1) * 8
    per_w = -(-n_chunks // NW)
    n_pairs = (per_w + 1) // 2
    mesh = plsc.VectorSubcoreMesh(core_axis_name="c", subcore_axis_name="s")

    @functools.partial(
        pl.kernel,
        mesh=mesh,
        compiler_params=pltpu.CompilerParams(use_tc_tiling_on_sc=True,
                                             needs_layout_passes=False),
        out_type=jax.ShapeDtypeStruct((vp4, 128), jnp.float32),
        scratch_types=[
            pltpu.VMEM((rows, D_MODEL), jnp.float32),
            pltpu.VMEM((rows, D_MODEL), jnp.float32),
            pltpu.VMEM((rows // 4, 128), jnp.float32),
            pltpu.VMEM((rows // 4, 128), jnp.float32),
            pltpu.VMEM((tail_rows, D_MODEL), jnp.float32),
            pltpu.VMEM((1, D_MODEL), jnp.float32),
            pltpu.VMEM((24, 128), jnp.float32),
            pltpu.SemaphoreType.DMA,
            pltpu.SemaphoreType.DMA,
        ],
    )
    def depad_kernel(table_hbm, out_hbm, rda, rdb, wra, wrb, rdt, rdt2,
                     wrt, semr, semw):
        wid = lax.axis_index("s") * NC + lax.axis_index("c")
        c0 = wid * per_w

        def bases(cid):
            base = pl.multiple_of(cid * rows, rows)
            ob = pl.multiple_of(cid * (rows // 4), rows // 4)
            return base, ob

        def rd(buf, cid, issue):
            base, _ = bases(cid)
            src = table_hbm.at[pl.ds(base, rows)]
            if issue:
                pltpu.async_copy(src, buf, semr)
            else:
                pltpu.make_async_copy(src, buf, semr).wait()

        def wr(buf, cid, issue):
            _, ob = bases(cid)
            dst = out_hbm.at[pl.ds(ob, rows // 4)]
            if issue:
                pltpu.async_copy(buf, dst, semw)
            else:
                pltpu.make_async_copy(buf, dst, semw).wait()

        def vcopy(rbuf, wbuf, nr):
            def p_body(p4, c):
                for q in range(4):
                    for h in range(2):
                        x = rbuf[p4 * 4 + q, pl.ds(h * L, L)]
                        wbuf[p4, pl.ds(q * D_MODEL + h * L, L)] = x
                return c

            lax.fori_loop(0, nr // 4, p_body, 0)

        def guarded(cid, fn):
            @pl.when(cid < n_chunks)
            def _():
                fn()

        def slot_cid(s):
            return wid + NW * s

        guarded(slot_cid(0), lambda: rd(rda, slot_cid(0), True))

        def pair_body(p, carry):
            ce = slot_cid(2 * p)
            co = slot_cid(2 * p + 1)
            cn = slot_cid(2 * p + 2)

            @pl.when(p > 0)
            def _():
                guarded(ce - 2 * NW, lambda: wr(wra, ce - 2 * NW, False))
                guarded(co - 2 * NW, lambda: wr(wrb, co - 2 * NW, False))

            guarded(ce, lambda: rd(rda, ce, False))
            guarded(co, lambda: rd(rdb, co, True))

            def do_even():
                vcopy(rda, wra, rows)
                wr(wra, ce, True)

            guarded(ce, do_even)
            guarded(co, lambda: rd(rdb, co, False))
            guarded(cn, lambda: rd(rda, cn, True))

            def do_odd():
                vcopy(rdb, wrb, rows)
                wr(wrb, co, True)

            guarded(co, do_odd)
            return carry

        lax.fori_loop(0, n_pairs, pair_body, 0)
        last_e = slot_cid(2 * n_pairs - 2)
        last_o = slot_cid(2 * n_pairs - 1)
        guarded(last_e, lambda: wr(wra, last_e, False))
        guarded(last_o, lambda: wr(wrb, last_o, False))

        @pl.when(wid == 0)
        def _():
            pltpu.sync_copy(table_hbm.at[pl.ds(tail_base, tail_rows)], rdt)
            pltpu.sync_copy(table_hbm.at[pl.ds(last_row, 1)], rdt2)
            vcopy(rdt, wrt, tail_rows)
            for h in range(2):
                wrt[tail_rows // 4, pl.ds(h * L, L)] = rdt2[0, pl.ds(h * L,
                                                                     L)]
            pltpu.sync_copy(wrt, out_hbm.at[pl.ds(tail_base // 4, 24)])

    return depad_kernel


@functools.lru_cache(maxsize=None)
def _make_gather(N: int, S: int):
    itiles_per_w = N // (NW * IT)
    n_jc = S // JC
    mesh = plsc.VectorSubcoreMesh(core_axis_name="c", subcore_axis_name="s")

    @functools.partial(
        pl.kernel,
        mesh=mesh,
        compiler_params=pltpu.CompilerParams(use_tc_tiling_on_sc=False,
                                             needs_layout_passes=False),
        out_type=jax.ShapeDtypeStruct((S * 4, N // IT, 8 * IT), jnp.float32),
        scratch_types=[
            pltpu.VMEM((JC, IT), jnp.int32),
            pltpu.VMEM((JC, IT), jnp.int32),
            pltpu.VMEM((JC * IT, D_MODEL), jnp.float32),
            pltpu.VMEM((JC * IT, D_MODEL), jnp.float32),
            pltpu.VMEM((JC * 4 * 8 * IT,), jnp.float32),
            pltpu.VMEM((JC * 4 * 8 * IT,), jnp.float32),
            pltpu.SemaphoreType.DMA,
            pltpu.SemaphoreType.DMA,
        ],
    )
    def gather_kernel(event_hbm, table_hbm, out_hbm, idx0, idx1, rows0,
                      rows1, st0, st1, semg, semo):
        wid = lax.axis_index("s") * NC + lax.axis_index("c")
        iota = lax.iota(jnp.int32, L)
        table_r = table_hbm

        def load_and_fire(idx_v, rows_v, itile_abs, jc):
            pltpu.sync_copy(
                event_hbm.at[pl.ds(jc * JC, JC),
                             pl.ds(itile_abs * IT, IT)],
                idx_v)
            for m in range(JC):
                pltpu.async_copy(table_r.at[idx_v.at[m]],
                                 rows_v.at[pl.ds(m * IT, IT)], semg)

        def drain_gathers(idx_v, rows_v):
            for m in range(JC):
                pltpu.make_async_copy(table_r.at[idx_v.at[m]],
                                      rows_v.at[pl.ds(m * IT, IT)],
                                      semg).wait()

        def transpose(rows_v, st_v):
            def j_body(j_loc, c):
                row_base = iota + j_loc * IT
                dst0 = ((j_loc * 4 + (iota >> 3)) << 10) \
                    + ((iota & 7) << 7) + iota

                def k_body(k0, carry):
                    col_v, dst_v = carry
                    for _ in range(2):
                        for ib in range(0, IT, L):
                            x = plsc.load_gather(rows_v,
                                                 [row_base + ib, col_v])
                            plsc.store_scatter(st_v, [dst_v + ib], x)
                        wrap = col_v == 31
                        dst_v = dst_v + jnp.where(wrap, 128 - 4096, 128)
                        col_v = (col_v + 1) & 31
                    return (col_v, dst_v)

                lax.fori_loop(0, D_MODEL // 2, k_body, (iota & 31, dst0))
                return c

            lax.fori_loop(0, JC, j_body, 0)

        def out_copy(st_v, itile_abs, jc, issue):
            for r in range(JC * 4):
                src = st_v.at[pl.ds(r * 8 * IT, 8 * IT)]
                dst = out_hbm.at[jc * JC * 4 + r, itile_abs]
                if issue:
                    pltpu.async_copy(src, dst, semo)
                else:
                    pltpu.make_async_copy(src, dst, semo).wait()

        def it_body(it, carry):
            itile_abs = wid * itiles_per_w + it
            load_and_fire(idx0, rows0, itile_abs, 0)

            def pair_body(p, carry2):
                e = 2 * p
                o = 2 * p + 1
                drain_gathers(idx0, rows0)
                load_and_fire(idx1, rows1, itile_abs, o)

                @pl.when(p > 0)
                def _():
                    out_copy(st0, itile_abs, e - 2, issue=False)

                transpose(rows0, st0)
                out_copy(st0, itile_abs, e, issue=True)

                drain_gathers(idx1, rows1)

                @pl.when(o + 1 < n_jc)
                def _():
                    load_and_fire(idx0, rows0, itile_abs, o + 1)

                @pl.when(p > 0)
                def _():
                    out_copy(st1, itile_abs, o - 2, issue=False)

                transpose(rows1, st1)
                out_copy(st1, itile_abs, o, issue=True)
                return carry2

            lax.fori_loop(0, n_jc // 2, pair_body, 0)
            out_copy(st0, itile_abs, n_jc - 2, issue=False)
            out_copy(st1, itile_abs, n_jc - 1, issue=False)
            return carry

        lax.fori_loop(0, itiles_per_w, it_body, 0)

    return gather_kernel


def kernel(event, table):
    n, s = event.shape
    table_pad = _make_depad(table.shape[0])(table)
    table_lin = table_pad.reshape(table_pad.shape[0] * 4, D_MODEL)
    out3 = _make_gather(n, s)(event.T, table_lin)
    out5 = out3.reshape(s, 4, n // IT, 8, IT)
    return out5.transpose(2, 4, 0, 1, 3).reshape(n, s, D_MODEL)


# speedup vs baseline: 1.0442x; 1.0442x over previous
import functools

import jax
import jax.numpy as jnp
from jax import lax
from jax.experimental import pallas as pl
from jax.experimental.pallas import tpu as pltpu
from jax.experimental.pallas import tpu_sc as plsc

D_MODEL = 32
NC = 2
NS = 16
NW = NC * NS
JC = 4
IT = 128
L = 16


@functools.lru_cache(maxsize=None)
def _make_gather(N: int, S: int):
    itiles_per_w = N // (NW * IT)
    n_jc = S // JC
    mesh = plsc.VectorSubcoreMesh(core_axis_name="c", subcore_axis_name="s")

    @functools.partial(
        pl.kernel,
        mesh=mesh,
        compiler_params=pltpu.CompilerParams(use_tc_tiling_on_sc=False,
                                             needs_layout_passes=False),
        out_type=jax.ShapeDtypeStruct((S * 4, N // IT, 8 * IT), jnp.float32),
        scratch_types=[
            pltpu.VMEM((JC, IT), jnp.int32),
            pltpu.VMEM((JC, IT), jnp.int32),
            pltpu.VMEM((JC * IT, D_MODEL), jnp.float32),
            pltpu.VMEM((JC * IT, D_MODEL), jnp.float32),
            pltpu.VMEM((JC * 4 * 8 * IT,), jnp.float32),
            pltpu.VMEM((JC * 4 * 8 * IT,), jnp.float32),
            pltpu.SemaphoreType.DMA,
            pltpu.SemaphoreType.DMA,
        ],
    )
    def gather_kernel(event_hbm, table_hbm, out_hbm, idx0, idx1, rows0,
                      rows1, st0, st1, semg, semo):
        wid = lax.axis_index("s") * NC + lax.axis_index("c")
        iota = lax.iota(jnp.int32, L)

        def load_and_fire(idx_v, rows_v, itile_abs, jc):
            pltpu.sync_copy(
                event_hbm.at[pl.ds(jc * JC, JC),
                             pl.ds(itile_abs * IT, IT)],
                idx_v)
            for m in range(JC):
                pltpu.async_copy(table_hbm.at[idx_v.at[m]],
                                 rows_v.at[pl.ds(m * IT, IT)], semg)

        def drain_gathers(idx_v, rows_v):
            for m in range(JC):
                pltpu.make_async_copy(table_hbm.at[idx_v.at[m]],
                                      rows_v.at[pl.ds(m * IT, IT)],
                                      semg).wait()

        def transpose(rows_v, st_v):
            def j_body(j_loc, c):
                row_base = iota + j_loc * IT
                dst0 = ((j_loc * 4 + (iota >> 3)) << 10) \
                    + ((iota & 7) << 7) + iota

                def k_body(k0, carry):
                    col_v, dst_v = carry
                    for _ in range(2):
                        for ib in range(0, IT, L):
                            x = plsc.load_gather(rows_v,
                                                 [row_base + ib, col_v])
                            plsc.store_scatter(st_v, [dst_v + ib], x)
                        wrap = col_v == 31
                        dst_v = dst_v + jnp.where(wrap, 128 - 4096, 128)
                        col_v = (col_v + 1) & 31
                    return (col_v, dst_v)

                lax.fori_loop(0, D_MODEL // 2, k_body, (iota & 31, dst0))
                return c

            lax.fori_loop(0, JC, j_body, 0)

        def out_copy(st_v, itile_abs, jc, issue):
            for r in range(JC * 4):
                src = st_v.at[pl.ds(r * 8 * IT, 8 * IT)]
                dst = out_hbm.at[jc * JC * 4 + r, itile_abs]
                if issue:
                    pltpu.async_copy(src, dst, semo)
                else:
                    pltpu.make_async_copy(src, dst, semo).wait()

        def it_body(it, carry):
            itile_abs = wid * itiles_per_w + it
            load_and_fire(idx0, rows0, itile_abs, 0)

            def pair_body(p, carry2):
                e = 2 * p
                o = 2 * p + 1
                drain_gathers(idx0, rows0)
                load_and_fire(idx1, rows1, itile_abs, o)

                @pl.when(p > 0)
                def _():
                    out_copy(st0, itile_abs, e - 2, issue=False)

                transpose(rows0, st0)
                out_copy(st0, itile_abs, e, issue=True)

                drain_gathers(idx1, rows1)

                @pl.when(o + 1 < n_jc)
                def _():
                    load_and_fire(idx0, rows0, itile_abs, o + 1)

                @pl.when(p > 0)
                def _():
                    out_copy(st1, itile_abs, o - 2, issue=False)

                transpose(rows1, st1)
                out_copy(st1, itile_abs, o, issue=True)
                return carry2

            lax.fori_loop(0, n_jc // 2, pair_body, 0)
            out_copy(st0, itile_abs, n_jc - 2, issue=False)
            out_copy(st1, itile_abs, n_jc - 1, issue=False)
            return carry

        lax.fori_loop(0, itiles_per_w, it_body, 0)

    return gather_kernel


def kernel(event, table):
    n, s = event.shape
    out3 = _make_gather(n, s)(event.T, table)
    out5 = out3.reshape(s, 4, n // IT, 8, IT)
    return out5.transpose(2, 4, 0, 1, 3).reshape(n, s, D_MODEL)
